# baseline (device time: 36177 ns/iter reference)
import jax
import jax.numpy as jnp
from jax import lax
from jax.experimental import pallas as pl
from jax.experimental.pallas import tpu as pltpu

N_DEV = 4
B_LOC = 2
SQ = 128
SKV = 128
HQ = 16
HG = HQ // N_DEV
DH = 64
D_MODEL = 512
DG = HG * DH
BLK = 64
WQ_H = D_MODEL // 2
WO_H = DG // 2


def kernel(x, Wq, K_ext, V_ext, Wo):
    def body(x_ref, wq_ref, k_hbm, v_hbm, wo_ref, out_ref,
             k_t, v_t, wq_l, wq_r, wq_d, wo_l, wo_r, wo_d,
             ctx_a, ctx_b, k_sems, v_sems, send_sems, recv_sems):
        my_pos = lax.axis_index("i")
        left = (my_pos + N_DEV - 1) % N_DEV
        right = (my_pos + 1) % N_DEV
        diag = (my_pos + 2) % N_DEV

        origins = (my_pos, left, right, diag)
        for s in range(N_DEV):
            for b in range(B_LOC):
                for hl in range(HG):
                    i = s * (B_LOC * HG) + b * HG + hl
                    bg = my_pos * B_LOC + b
                    g = origins[s] * HG + hl
                    pltpu.make_async_copy(
                        k_hbm.at[bg, :, g, :], k_t.at[i], k_sems.at[i]
                    ).start()
                    pltpu.make_async_copy(
                        v_hbm.at[bg, :, g, :], v_t.at[i], v_sems.at[i]
                    ).start()

        barrier_sem = pltpu.get_barrier_semaphore()
        for nbr in (left, right):
            pl.semaphore_signal(
                barrier_sem, inc=1,
                device_id=(nbr,), device_id_type=pl.DeviceIdType.MESH,
            )
        pl.semaphore_wait(barrier_sem, 2)

        def rdma(i, src, dst, dev):
            return pltpu.make_async_remote_copy(
                src_ref=src, dst_ref=dst,
                send_sem=send_sems.at[i], recv_sem=recv_sems.at[i],
                device_id=(dev,), device_id_type=pl.DeviceIdType.MESH,
            )

        d0 = rdma(0, wq_ref, wq_l, right)
        d2 = rdma(2, wq_ref, wq_r, left)
        d1 = rdma(1, wo_ref, wo_l, right)
        d3 = rdma(3, wo_ref, wo_r, left)
        d0.start()
        d2.start()
        d1.start()
        d3.start()

        xf = x_ref[...].reshape(B_LOC * SQ, D_MODEL)

        def wait_kv(s):
            for b in range(B_LOC):
                for hl in range(HG):
                    i = s * (B_LOC * HG) + b * HG + hl
                    pltpu.make_async_copy(
                        k_hbm.at[0, :, 0, :], k_t.at[i], k_sems.at[i]
                    ).wait()
                    pltpu.make_async_copy(
                        v_hbm.at[0, :, 0, :], v_t.at[i], v_sems.at[i]
                    ).wait()

        def attn(s, q, ctx_ref):
            for b in range(B_LOC):
                for hl in range(HG):
                    i = s * (B_LOC * HG) + b * HG + hl
                    kh = k_t[i]
                    vh = v_t[i]
                    qh = q[b * SQ:(b + 1) * SQ, hl * DH:(hl + 1) * DH]
                    sa = lax.dot_general(
                        qh[0:BLK], kh[0:BLK], (((1,), (1,)), ((), ())),
                        preferred_element_type=jnp.float32,
                    )
                    ea = jnp.exp(sa)
                    ra = 1.0 / jnp.sum(ea, axis=1, keepdims=True)
                    ctxa = jnp.dot(
                        ea, vh[0:BLK], preferred_element_type=jnp.float32
                    ) * ra
                    sb = lax.dot_general(
                        qh[BLK:SQ], kh, (((1,), (1,)), ((), ())),
                        preferred_element_type=jnp.float32,
                    )
                    eb = jnp.exp(sb)
                    rb = 1.0 / jnp.sum(eb, axis=1, keepdims=True)
                    ctxb = jnp.dot(
                        eb, vh, preferred_element_type=jnp.float32
                    ) * rb
                    r0 = b * SQ
                    c0 = hl * DH
                    ctx_ref[r0:r0 + BLK, c0:c0 + DH] = ctxa
                    ctx_ref[r0 + BLK:r0 + SQ, c0:c0 + DH] = ctxb

        def qdot(wq_g):
            return jnp.dot(
                xf, wq_g, preferred_element_type=jnp.float32
            ) * 0.125

        def outdot(ctx_ref, wo_g, first=False):
            partial = jnp.dot(
                ctx_ref[...], wo_g, preferred_element_type=jnp.float32
            ).reshape(B_LOC, SQ, D_MODEL)
            if first:
                out_ref[...] = partial
            else:
                out_ref[...] = out_ref[...] + partial

        wait_kv(0)
        attn(0, qdot(wq_ref[...]), ctx_a)
        outdot(ctx_a, wo_ref[...], first=True)

        d0.wait_recv()
        d4 = rdma(4, wq_l.at[pl.ds(0, WQ_H)], wq_d.at[pl.ds(0, WQ_H)], right)
        d4.start()
        wait_kv(1)
        attn(1, qdot(wq_l[...]), ctx_a)

        d2.wait_recv()
        d6 = rdma(6, wq_r.at[pl.ds(WQ_H, WQ_H)], wq_d.at[pl.ds(WQ_H, WQ_H)], left)
        d6.start()
        wait_kv(2)
        attn(2, qdot(wq_r[...]), ctx_b)

        d1.wait_recv()
        d5 = rdma(5, wo_l.at[pl.ds(0, WO_H)], wo_d.at[pl.ds(0, WO_H)], right)
        d5.start()
        outdot(ctx_a, wo_l[...])
        d3.wait_recv()
        d7 = rdma(7, wo_r.at[pl.ds(WO_H, WO_H)], wo_d.at[pl.ds(WO_H, WO_H)], left)
        d7.start()
        outdot(ctx_b, wo_r[...])

        d4.wait_recv()
        d6.wait_recv()
        wait_kv(3)
        attn(3, qdot(wq_d[...]), ctx_a)
        d5.wait_recv()
        d7.wait_recv()
        outdot(ctx_a, wo_d[...])

        for d in (d0, d1, d2, d3, d4, d5, d6, d7):
            d.wait_send()

    grp = B_LOC * HQ
    return pl.pallas_call(
        body,
        out_shape=jax.ShapeDtypeStruct((B_LOC, SQ, D_MODEL), jnp.float32),
        in_specs=[
            pl.BlockSpec(memory_space=pltpu.VMEM),
            pl.BlockSpec(memory_space=pltpu.VMEM),
            pl.BlockSpec(memory_space=pltpu.MemorySpace.HBM),
            pl.BlockSpec(memory_space=pltpu.MemorySpace.HBM),
            pl.BlockSpec(memory_space=pltpu.VMEM),
        ],
        out_specs=pl.BlockSpec(memory_space=pltpu.VMEM),
        scratch_shapes=[
            pltpu.VMEM((grp, SKV, DH), jnp.float32),
            pltpu.VMEM((grp, SKV, DH), jnp.float32),
            pltpu.VMEM((D_MODEL, DG), jnp.float32),
            pltpu.VMEM((D_MODEL, DG), jnp.float32),
            pltpu.VMEM((D_MODEL, DG), jnp.float32),
            pltpu.VMEM((DG, D_MODEL), jnp.float32),
            pltpu.VMEM((DG, D_MODEL), jnp.float32),
            pltpu.VMEM((DG, D_MODEL), jnp.float32),
            pltpu.VMEM((B_LOC * SQ, DG), jnp.float32),
            pltpu.VMEM((B_LOC * SQ, DG), jnp.float32),
            pltpu.SemaphoreType.DMA((grp,)),
            pltpu.SemaphoreType.DMA((grp,)),
            pltpu.SemaphoreType.DMA((8,)),
            pltpu.SemaphoreType.DMA((8,)),
        ],
        compiler_params=pltpu.CompilerParams(collective_id=0),
    )(x, Wq, K_ext, V_ext, Wo)


# device time: 29333 ns/iter; 1.2333x vs baseline; 1.2333x over previous
import jax
import jax.numpy as jnp
from jax import lax
from jax.experimental import pallas as pl
from jax.experimental.pallas import tpu as pltpu

N_DEV = 4
B_LOC = 2
SQ = 128
SKV = 128
HQ = 16
HG = HQ // N_DEV
DH = 64
D_MODEL = 512
DG = HG * DH
BLK = 64
WQ_H = D_MODEL // 2
WO_H = DG // 2


def kernel(x, Wq, K_ext, V_ext, Wo):
    K2 = K_ext.reshape(N_DEV * B_LOC, SKV, HQ * DH)
    V2 = V_ext.reshape(N_DEV * B_LOC, SKV, HQ * DH)

    def body(x_ref, wq_ref, k_hbm, v_hbm, wo_ref, out_ref,
             k_t, v_t, wq_l, wq_r, wq_d, wo_l, wo_r, wo_d,
             ctx_a, ctx_b, k_sems, v_sems, send_sems, recv_sems):
        my_pos = lax.axis_index("i")
        left = (my_pos + N_DEV - 1) % N_DEV
        right = (my_pos + 1) % N_DEV
        diag = (my_pos + 2) % N_DEV

        origins = (my_pos, left, right, diag)
        bg0 = my_pos * B_LOC
        for s in range(N_DEV):
            c = origins[s] * DG
            pltpu.make_async_copy(
                k_hbm.at[pl.ds(bg0, B_LOC), :, pl.ds(c, DG)],
                k_t.at[s], k_sems.at[s],
            ).start()
            pltpu.make_async_copy(
                v_hbm.at[pl.ds(bg0, B_LOC), :, pl.ds(c, DG)],
                v_t.at[s], v_sems.at[s],
            ).start()

        barrier_sem = pltpu.get_barrier_semaphore()
        for nbr in (left, right):
            pl.semaphore_signal(
                barrier_sem, inc=1,
                device_id=(nbr,), device_id_type=pl.DeviceIdType.MESH,
            )
        pl.semaphore_wait(barrier_sem, 2)

        def rdma(i, src, dst, dev):
            return pltpu.make_async_remote_copy(
                src_ref=src, dst_ref=dst,
                send_sem=send_sems.at[i], recv_sem=recv_sems.at[i],
                device_id=(dev,), device_id_type=pl.DeviceIdType.MESH,
            )

        d0 = rdma(0, wq_ref, wq_l, right)
        d2 = rdma(2, wq_ref, wq_r, left)
        d1 = rdma(1, wo_ref, wo_l, right)
        d3 = rdma(3, wo_ref, wo_r, left)
        d0.start()
        d2.start()
        d1.start()
        d3.start()

        xf = x_ref[...].reshape(B_LOC * SQ, D_MODEL)

        def wait_kv(s):
            pltpu.make_async_copy(
                k_hbm.at[pl.ds(0, B_LOC), :, pl.ds(0, DG)],
                k_t.at[s], k_sems.at[s],
            ).wait()
            pltpu.make_async_copy(
                v_hbm.at[pl.ds(0, B_LOC), :, pl.ds(0, DG)],
                v_t.at[s], v_sems.at[s],
            ).wait()

        def attn(s, q, ctx_ref):
            for b in range(B_LOC):
                for hl in range(HG):
                    kh = k_t[s, b, :, hl * DH:(hl + 1) * DH]
                    vh = v_t[s, b, :, hl * DH:(hl + 1) * DH]
                    qh = q[b * SQ:(b + 1) * SQ, hl * DH:(hl + 1) * DH]
                    sa = lax.dot_general(
                        qh[0:BLK], kh[0:BLK], (((1,), (1,)), ((), ())),
                        preferred_element_type=jnp.float32,
                    )
                    ea = jnp.exp(sa)
                    ra = 1.0 / jnp.sum(ea, axis=1, keepdims=True)
                    ctxa = jnp.dot(
                        ea, vh[0:BLK], preferred_element_type=jnp.float32
                    ) * ra
                    sb = lax.dot_general(
                        qh[BLK:SQ], kh, (((1,), (1,)), ((), ())),
                        preferred_element_type=jnp.float32,
                    )
                    eb = jnp.exp(sb)
                    rb = 1.0 / jnp.sum(eb, axis=1, keepdims=True)
                    ctxb = jnp.dot(
                        eb, vh, preferred_element_type=jnp.float32
                    ) * rb
                    r0 = b * SQ
                    c0 = hl * DH
                    ctx_ref[r0:r0 + BLK, c0:c0 + DH] = ctxa
                    ctx_ref[r0 + BLK:r0 + SQ, c0:c0 + DH] = ctxb

        def qdot(wq_g):
            return jnp.dot(
                xf, wq_g, preferred_element_type=jnp.float32
            ) * 0.125

        def outdot(ctx_ref, wo_g, first=False):
            partial = jnp.dot(
                ctx_ref[...], wo_g, preferred_element_type=jnp.float32
            ).reshape(B_LOC, SQ, D_MODEL)
            if first:
                out_ref[...] = partial
            else:
                out_ref[...] = out_ref[...] + partial

        wait_kv(0)
        attn(0, qdot(wq_ref[...]), ctx_a)
        outdot(ctx_a, wo_ref[...], first=True)

        d0.wait_recv()
        d4 = rdma(4, wq_l.at[pl.ds(0, WQ_H)], wq_d.at[pl.ds(0, WQ_H)], right)
        d4.start()
        wait_kv(1)
        attn(1, qdot(wq_l[...]), ctx_a)

        d2.wait_recv()
        d6 = rdma(6, wq_r.at[pl.ds(WQ_H, WQ_H)], wq_d.at[pl.ds(WQ_H, WQ_H)], left)
        d6.start()
        wait_kv(2)
        attn(2, qdot(wq_r[...]), ctx_b)

        d1.wait_recv()
        d5 = rdma(5, wo_l.at[pl.ds(0, WO_H)], wo_d.at[pl.ds(0, WO_H)], right)
        d5.start()
        outdot(ctx_a, wo_l[...])
        d3.wait_recv()
        d7 = rdma(7, wo_r.at[pl.ds(WO_H, WO_H)], wo_d.at[pl.ds(WO_H, WO_H)], left)
        d7.start()
        outdot(ctx_b, wo_r[...])

        d4.wait_recv()
        d6.wait_recv()
        wait_kv(3)
        attn(3, qdot(wq_d[...]), ctx_a)
        d5.wait_recv()
        d7.wait_recv()
        outdot(ctx_a, wo_d[...])

        for d in (d0, d1, d2, d3, d4, d5, d6, d7):
            d.wait_send()

    return pl.pallas_call(
        body,
        out_shape=jax.ShapeDtypeStruct((B_LOC, SQ, D_MODEL), jnp.float32),
        in_specs=[
            pl.BlockSpec(memory_space=pltpu.VMEM),
            pl.BlockSpec(memory_space=pltpu.VMEM),
            pl.BlockSpec(memory_space=pltpu.MemorySpace.HBM),
            pl.BlockSpec(memory_space=pltpu.MemorySpace.HBM),
            pl.BlockSpec(memory_space=pltpu.VMEM),
        ],
        out_specs=pl.BlockSpec(memory_space=pltpu.VMEM),
        scratch_shapes=[
            pltpu.VMEM((N_DEV, B_LOC, SKV, DG), jnp.float32),
            pltpu.VMEM((N_DEV, B_LOC, SKV, DG), jnp.float32),
            pltpu.VMEM((D_MODEL, DG), jnp.float32),
            pltpu.VMEM((D_MODEL, DG), jnp.float32),
            pltpu.VMEM((D_MODEL, DG), jnp.float32),
            pltpu.VMEM((DG, D_MODEL), jnp.float32),
            pltpu.VMEM((DG, D_MODEL), jnp.float32),
            pltpu.VMEM((DG, D_MODEL), jnp.float32),
            pltpu.VMEM((B_LOC * SQ, DG), jnp.float32),
            pltpu.VMEM((B_LOC * SQ, DG), jnp.float32),
            pltpu.SemaphoreType.DMA((N_DEV,)),
            pltpu.SemaphoreType.DMA((N_DEV,)),
            pltpu.SemaphoreType.DMA((8,)),
            pltpu.SemaphoreType.DMA((8,)),
        ],
        compiler_params=pltpu.CompilerParams(collective_id=0),
    )(x, Wq, K2, V2, Wo)
